# baseline (speedup 1.0000x reference)
import functools

import jax
import jax.numpy as jnp
from jax.experimental import pallas as pl
from jax.experimental.pallas import tpu as pltpu

_K = 16
_NPTS_DS = 1024
_ROWS = 256


def _tree16(vals):
    s = [vals[t] + vals[t + 8] for t in range(8)]
    u = [s[t] + s[t + 4] for t in range(4)]
    w = [u[t] + u[t + 2] for t in range(2)]
    return w[0] + w[1]


def _score_body(xb_ref, xr_ref, xt_ref, score_ref):
    xb = xb_ref[0]
    xr = xr_ref[0]
    xt = xt_ref[0]
    n = xb.shape[1]
    r = xr.shape[1]

    sq_c = (xb[0] * xb[0] + xb[1] * xb[1]) + xb[2] * xb[2]
    sq_r = (xr[0] * xr[0] + xr[1] * xr[1]) + xr[2] * xr[2]

    inner = jax.lax.dot_general(
        xr, xb, (((0,), (0,)), ((), ())),
        preferred_element_type=jnp.float32)

    neg = (2.0 * inner - sq_r[:, None]) - sq_c[None, :]

    a16 = xt.astype(jnp.bfloat16)
    r1 = xt - a16.astype(jnp.float32)
    b16 = r1.astype(jnp.bfloat16)
    r2 = r1 - b16.astype(jnp.float32)
    c16 = r2.astype(jnp.bfloat16)
    abc = jnp.concatenate([a16, b16, c16], axis=1)

    iota = jax.lax.broadcasted_iota(jnp.int32, (r, n), 1)
    vals = []
    for _ in range(_K):
        m = jnp.max(neg, axis=1, keepdims=True)
        cand = jnp.where(neg == m, iota, n)
        j = jnp.min(cand, axis=1, keepdims=True)
        sel = iota == j
        neg = jnp.where(sel, -jnp.inf, neg)
        onehot = jnp.where(sel, 1.0, 0.0).astype(jnp.bfloat16)
        picked = jax.lax.dot_general(
            onehot, abc, (((1,), (0,)), ((), ())),
            preferred_element_type=jnp.float32)
        vals.append((picked[:, 0:3] + picked[:, 3:6]) + picked[:, 6:9])

    mean = _tree16(vals) * jnp.float32(0.0625)
    sqs = [(v - mean) * (v - mean) for v in vals]
    var = _tree16(sqs) * jnp.float32(1.0 / 15.0)
    std = jnp.sqrt(var)
    score = (std[:, 0] + std[:, 2]) + std[:, 1]
    score_ref[0, 0] = score


def _point_scores(x):
    b, _, n = x.shape
    xt = jnp.swapaxes(x, 1, 2)
    nb = n // _ROWS
    grid = (b, nb)
    out = pl.pallas_call(
        _score_body,
        grid=grid,
        in_specs=[
            pl.BlockSpec((1, 3, n), lambda bi, ri: (bi, 0, 0)),
            pl.BlockSpec((1, 3, _ROWS), lambda bi, ri: (bi, 0, ri)),
            pl.BlockSpec((1, n, 3), lambda bi, ri: (bi, 0, 0)),
        ],
        out_specs=pl.BlockSpec(
            (1, 1, _ROWS), lambda bi, ri: (bi * nb + ri, 0, 0)),
        out_shape=jax.ShapeDtypeStruct((b * nb, 1, _ROWS), jnp.float32),
    )(x, x, xt)
    return out.reshape(b, n)


_IBLK = 1024


def _split3(v):
    a = v.astype(jnp.bfloat16)
    r1 = v - a.astype(jnp.float32)
    b = r1.astype(jnp.bfloat16)
    r2 = r1 - b.astype(jnp.float32)
    c = r2.astype(jnp.bfloat16)
    return jnp.concatenate([a, b, c], axis=0)


def _select_body(score_ref, x_ref, y_ref, xyz_ref, pts_ref):
    ri = pl.program_id(1)
    n = score_ref.shape[2]
    ib = x_ref.shape[2]
    base = ri * ib

    s_row = score_ref[0]
    si = score_ref[0, 0, pl.ds(base, ib)].reshape(ib, 1)
    jdx = jax.lax.broadcasted_iota(jnp.int32, (1, n), 1)
    idx_i = base + jax.lax.broadcasted_iota(jnp.int32, (ib, 1), 0)
    before = (s_row > si) | ((s_row == si) & (jdx < idx_i))
    rank = jnp.sum(before.astype(jnp.int32), axis=1)

    pio = jax.lax.broadcasted_iota(jnp.int32, (_NPTS_DS, ib), 0)
    oh = jnp.where(pio == rank[None, :], 1.0, 0.0).astype(jnp.bfloat16)

    xs = _split3(x_ref[0])
    ys = _split3(y_ref[0])
    px = jax.lax.dot_general(xs, oh, (((1,), (1,)), ((), ())),
                             preferred_element_type=jnp.float32)
    py = jax.lax.dot_general(ys, oh, (((1,), (1,)), ((), ())),
                             preferred_element_type=jnp.float32)
    gx = (px[0:3] + px[3:6]) + px[6:9]
    cy = y_ref.shape[1]
    gy = (py[0:cy] + py[cy:2 * cy]) + py[2 * cy:3 * cy]

    @pl.when(ri == 0)
    def _init():
        xyz_ref[0] = gx
        pts_ref[0] = gy

    @pl.when(ri != 0)
    def _acc():
        xyz_ref[0] += gx
        pts_ref[0] += gy


def _select_topk(score, x, y):
    b, _, n = x.shape
    cy = y.shape[1]
    nbi = n // _IBLK
    out = pl.pallas_call(
        _select_body,
        grid=(b, nbi),
        in_specs=[
            pl.BlockSpec((1, 1, n), lambda bi, ri: (bi, 0, 0)),
            pl.BlockSpec((1, 3, _IBLK), lambda bi, ri: (bi, 0, ri)),
            pl.BlockSpec((1, cy, _IBLK), lambda bi, ri: (bi, 0, ri)),
        ],
        out_specs=[
            pl.BlockSpec((1, 3, _NPTS_DS), lambda bi, ri: (bi, 0, 0)),
            pl.BlockSpec((1, cy, _NPTS_DS), lambda bi, ri: (bi, 0, 0)),
        ],
        out_shape=[
            jax.ShapeDtypeStruct((b, 3, _NPTS_DS), jnp.float32),
            jax.ShapeDtypeStruct((b, cy, _NPTS_DS), jnp.float32),
        ],
    )(score.reshape(b, 1, n), x, y)
    return out


def kernel(x, y):
    score = _point_scores(x)
    top_k_xyz, top_k_points = _select_topk(score, x, y)
    return (top_k_xyz, top_k_points)
